# baseline (speedup 1.0000x reference)
import functools

import jax
import jax.numpy as jnp
import numpy as np
from jax import lax
from jax.experimental import pallas as pl
from jax.experimental.pallas import tpu as pltpu
from jax.experimental.pallas import tpu_sc as plsc

MARGIN = 1.0
TAU = 0.1
NUM_NEG = 10
MN_LIM = -100.0

B = 4096
L = 4096
NC = 2
NS = 16
LANES = 16
NW = NC * NS
RPW = B // NW
GROUPS = RPW // LANES
CS = 1024
NCH = L // CS
CSP = CS + 1


def _sc_body(out_hbm, gt_hbm, part_hbm, bufs, sems, accv):
    wid = lax.axis_index("s") * NC + lax.axis_index("c")
    iota = lax.iota(jnp.int32, LANES)
    neg_inf = jnp.float32(-jnp.inf)
    acc = jnp.zeros((LANES,), jnp.float32)
    HS = CS // 2
    steps = [(g, ch) for g in range(GROUPS) for ch in range(NCH)]

    def issue(step_idx):
        g, ch = steps[step_idx]
        rowbase = wid * RPW + g * LANES
        c0 = ch * CS
        hs = []
        for src, dst, sem in zip(
            (out_hbm, gt_hbm), bufs[step_idx % 2], sems[step_idx % 2]
        ):
            hs.append(
                pltpu.async_copy(
                    src.at[pl.ds(rowbase, LANES), pl.ds(c0, CS)],
                    dst.at[:, pl.ds(0, CS)],
                    sem,
                )
            )
        return hs

    def stream_init():
        bg = jnp.full((LANES,), neg_inf, jnp.float32)
        bp = jnp.zeros((LANES,), jnp.float32)
        ms = [jnp.full((LANES,), neg_inf, jnp.float32) for _ in range(NUM_NEG)]
        return [bg, bp] + ms

    NST = 2 + NUM_NEG
    handles = issue(0)
    sa = sb = None
    for s, (g, ch) in enumerate(steps):
        if ch == 0:
            sa = stream_init()
            sb = stream_init()
        nxt = issue(s + 1) if s + 1 < len(steps) else None
        for h in handles:
            h.wait()
        handles = nxt
        ov, gv = bufs[s % 2]

        def upd_stream(cvec, st):
            bg, bp = st[0], st[1]
            ms_ = st[2:]
            o = plsc.load_gather(ov, [iota, cvec])
            gt = plsc.load_gather(gv, [iota, cvec])
            upd = gt > bg
            bg = jnp.where(upd, gt, bg)
            bp = jnp.where(upd, o, bp)
            x = jnp.where(gt >= 0.0, jnp.float32(MN_LIM), o)
            out = [bg, bp]
            for m in ms_:
                out.append(jnp.maximum(m, x))
                x = jnp.minimum(m, x)
            return out

        def body(c, carry):
            cva, cvb = carry[0], carry[1]
            na = upd_stream(cva, carry[2 : 2 + NST])
            nb = upd_stream(cvb, carry[2 + NST :])
            return (cva + 1, cvb + 1, *na, *nb)

        cva0 = jnp.zeros((LANES,), jnp.int32)
        cvb0 = jnp.full((LANES,), HS, jnp.int32)
        res = lax.fori_loop(0, HS, body, (cva0, cvb0, *sa, *sb), unroll=3)
        sa = list(res[2 : 2 + NST])
        sb = list(res[2 + NST :])
        if ch != NCH - 1:
            continue
        bga, bpa = sa[0], sa[1]
        bgb, bpb = sb[0], sb[1]
        updb = bgb > bga
        best_p = jnp.where(updb, bpb, bpa)
        ms = sa[2:]
        for xb in sb[2:]:
            x = xb
            nms = []
            for m in ms:
                nms.append(jnp.maximum(m, x))
                x = jnp.minimum(m, x)
            ms = nms
        zero = jnp.zeros((LANES,), jnp.float32)
        losses = [jnp.maximum(zero, m - best_p + jnp.float32(MARGIN)) for m in ms]
        zs = [jnp.where(l == 0.0, jnp.float32(MN_LIM), m) * jnp.float32(1.0 / TAU)
              for l, m in zip(losses, ms)]
        zm = zs[0]
        for z in zs[1:]:
            zm = jnp.maximum(zm, z)
        es = [jnp.exp(z - zm) for z in zs]
        ssum = es[0]
        for e in es[1:]:
            ssum = ssum + e
        contrib = zero
        for l, e in zip(losses, es):
            contrib = contrib + l * e
        acc = acc + contrib / ssum
    accv[...] = acc
    pltpu.sync_copy(accv, part_hbm.at[wid])


@functools.partial(
    pl.kernel,
    out_type=jax.ShapeDtypeStruct((NW, LANES), jnp.float32),
    mesh=plsc.VectorSubcoreMesh(
        core_axis_name="c", subcore_axis_name="s", num_cores=NC, num_subcores=NS
    ),
    scratch_types=(
        [pltpu.VMEM((LANES, CSP), jnp.float32) for _ in range(4)]
        + [pltpu.VMEM((LANES,), jnp.float32)]
        + [pltpu.SemaphoreType.DMA for _ in range(4)]
    ),
    compiler_params=pltpu.CompilerParams(
        use_tc_tiling_on_sc=False, needs_layout_passes=False
    ),
)
def _ohnm_sc(
    out_hbm, gt_hbm, part_hbm,
    ov0, gv0, ov1, gv1, accv,
    sm0, sm1, sm2, sm3,
):
    _sc_body(
        out_hbm, gt_hbm, part_hbm,
        [(ov0, gv0), (ov1, gv1)],
        [(sm0, sm1), (sm2, sm3)],
        accv,
    )


_GUM = np.asarray(
    jax.random.bits(jax.random.key(1), (B, L), jnp.uint32) >> 9
).astype(np.float32)


def kernel(output, target):
    gt = jnp.where(target > 0.0, jnp.asarray(_GUM), jnp.float32(-1.0))
    part = _ohnm_sc(output, gt)
    return jnp.sum(part) / jnp.float32(B * NUM_NEG)


# speedup vs baseline: 1.0090x; 1.0090x over previous
import functools

import jax
import jax.numpy as jnp
import numpy as np
from jax import lax
from jax.experimental import pallas as pl
from jax.experimental.pallas import tpu as pltpu
from jax.experimental.pallas import tpu_sc as plsc

MARGIN = 1.0
TAU = 0.1
NUM_NEG = 10
MN_LIM = -100.0

B = 4096
L = 4096
NC = 2
NS = 16
LANES = 16
NW = NC * NS
RPW = B // NW
GROUPS = RPW // LANES
CS = 1024
NCH = L // CS
CSP = CS + 1


def _sc_body(out_hbm, gt_hbm, part_hbm, bufs, sems, accv):
    wid = lax.axis_index("s") * NC + lax.axis_index("c")
    iota = lax.iota(jnp.int32, LANES)
    neg_inf = jnp.float32(-jnp.inf)
    acc = jnp.zeros((LANES,), jnp.float32)
    HS = CS // 2
    steps = [(g, ch) for g in range(GROUPS) for ch in range(NCH)]

    def issue(step_idx):
        g, ch = steps[step_idx]
        rowbase = wid * RPW + g * LANES
        c0 = ch * CS
        hs = []
        for src, dst, sem in zip(
            (out_hbm, gt_hbm), bufs[step_idx % 2], sems[step_idx % 2]
        ):
            hs.append(
                pltpu.async_copy(
                    src.at[pl.ds(rowbase, LANES), pl.ds(c0, CS)],
                    dst.at[:, pl.ds(0, CS)],
                    sem,
                )
            )
        return hs

    def stream_init():
        bg = jnp.full((LANES,), neg_inf, jnp.float32)
        bp = jnp.zeros((LANES,), jnp.float32)
        ms = [jnp.full((LANES,), neg_inf, jnp.float32) for _ in range(NUM_NEG)]
        return [bg, bp] + ms

    NST = 2 + NUM_NEG
    handles = issue(0)
    sa = sb = None
    for s, (g, ch) in enumerate(steps):
        if ch == 0:
            sa = stream_init()
            sb = stream_init()
        nxt = issue(s + 1) if s + 1 < len(steps) else None
        for h in handles:
            h.wait()
        handles = nxt
        ov, gv = bufs[s % 2]

        def upd_stream(cvec, st):
            bg, bp = st[0], st[1]
            ms_ = st[2:]
            o = plsc.load_gather(ov, [iota, cvec])
            gt = plsc.load_gather(gv, [iota, cvec])
            upd = gt > bg
            bg = jnp.where(upd, gt, bg)
            bp = jnp.where(upd, o, bp)
            x = jnp.where(gt >= 0.0, jnp.float32(MN_LIM), o)
            out = [bg, bp]
            for m in ms_:
                out.append(jnp.maximum(m, x))
                x = jnp.minimum(m, x)
            return out

        def body(c, carry):
            cva, cvb = carry[0], carry[1]
            na = upd_stream(cva, carry[2 : 2 + NST])
            nb = upd_stream(cvb, carry[2 + NST :])
            return (cva + 1, cvb + 1, *na, *nb)

        cva0 = jnp.zeros((LANES,), jnp.int32)
        cvb0 = jnp.full((LANES,), HS, jnp.int32)
        res = lax.fori_loop(0, HS, body, (cva0, cvb0, *sa, *sb), unroll=2)
        sa = list(res[2 : 2 + NST])
        sb = list(res[2 + NST :])
        if ch != NCH - 1:
            continue
        bga, bpa = sa[0], sa[1]
        bgb, bpb = sb[0], sb[1]
        updb = bgb > bga
        best_p = jnp.where(updb, bpb, bpa)
        ms = sa[2:]
        for xb in sb[2:]:
            x = xb
            nms = []
            for m in ms:
                nms.append(jnp.maximum(m, x))
                x = jnp.minimum(m, x)
            ms = nms
        zero = jnp.zeros((LANES,), jnp.float32)
        losses = [jnp.maximum(zero, m - best_p + jnp.float32(MARGIN)) for m in ms]
        zs = [jnp.where(l == 0.0, jnp.float32(MN_LIM), m) * jnp.float32(1.0 / TAU)
              for l, m in zip(losses, ms)]
        zm = zs[0]
        for z in zs[1:]:
            zm = jnp.maximum(zm, z)
        es = [jnp.exp(z - zm) for z in zs]
        ssum = es[0]
        for e in es[1:]:
            ssum = ssum + e
        contrib = zero
        for l, e in zip(losses, es):
            contrib = contrib + l * e
        acc = acc + contrib / ssum
    accv[...] = acc
    pltpu.sync_copy(accv, part_hbm.at[wid])


@functools.partial(
    pl.kernel,
    out_type=jax.ShapeDtypeStruct((NW, LANES), jnp.float32),
    mesh=plsc.VectorSubcoreMesh(
        core_axis_name="c", subcore_axis_name="s", num_cores=NC, num_subcores=NS
    ),
    scratch_types=(
        [pltpu.VMEM((LANES, CSP), jnp.float32) for _ in range(4)]
        + [pltpu.VMEM((LANES,), jnp.float32)]
        + [pltpu.SemaphoreType.DMA for _ in range(4)]
    ),
    compiler_params=pltpu.CompilerParams(
        use_tc_tiling_on_sc=False, needs_layout_passes=False
    ),
)
def _ohnm_sc(
    out_hbm, gt_hbm, part_hbm,
    ov0, gv0, ov1, gv1, accv,
    sm0, sm1, sm2, sm3,
):
    _sc_body(
        out_hbm, gt_hbm, part_hbm,
        [(ov0, gv0), (ov1, gv1)],
        [(sm0, sm1), (sm2, sm3)],
        accv,
    )


_GUM = np.asarray(
    jax.random.bits(jax.random.key(1), (B, L), jnp.uint32) >> 9
).astype(np.float32)


def kernel(output, target):
    gt = jnp.where(target > 0.0, jnp.asarray(_GUM), jnp.float32(-1.0))
    part = _ohnm_sc(output, gt)
    return jnp.sum(part) / jnp.float32(B * NUM_NEG)
